# baseline (speedup 1.0000x reference)
import math

import jax
import jax.numpy as jnp
from jax.experimental import pallas as pl
from jax.experimental.pallas import tpu as pltpu

_SIZE = 32000
_PAD = 0
_SMOOTH = 0.1
_CONF = 1.0 - _SMOOTH
_FILL = _SMOOTH / (_SIZE - 2)
_C0 = _CONF * math.log(_CONF) + _SMOOTH * math.log(_FILL)
_K = _CONF / _FILL

_BLK = 3200


def _body(t_ref, x_ref, out_ref, acc_ref):
    j = pl.program_id(0)
    nj = pl.num_programs(0)
    x = x_ref[...]
    n, blk = x.shape
    t = t_ref[:, 0]
    off = j * blk

    p = jnp.zeros((n, 128), jnp.float32)
    for k in range(blk // 128):
        xs = x[:, k * 128:(k + 1) * 128]
        cid = off + k * 128 + jax.lax.broadcasted_iota(jnp.int32, (n, 128), 1)
        z = jnp.where(cid == t[:, None], _K * xs, xs)
        if k == 0:
            z = jnp.where(cid == 0, 0.0, z)
        p = p + z

    @pl.when(j == 0)
    def _():
        acc_ref[...] = p
        out_ref[...] = jnp.zeros((1, 1), jnp.float32)

    @pl.when(j > 0)
    def _():
        acc_ref[...] += p

    @pl.when(j == nj - 1)
    def _():
        validf = (t != _PAD).astype(jnp.float32)
        ones = jnp.ones((128, 1), jnp.float32)
        rowz = jax.lax.dot(acc_ref[...], ones,
                           preferred_element_type=jnp.float32)[:, 0]
        nv = jnp.sum(validf)
        total = nv * _C0 - _FILL * jnp.sum(validf * rowz)
        out_ref[...] = total.reshape(1, 1)


@jax.jit
def kernel(x, target):
    n, size = x.shape
    t2 = target.reshape(n, 1)
    grid = size // _BLK
    out = pl.pallas_call(
        _body,
        grid=(grid,),
        in_specs=[
            pl.BlockSpec((n, 1), lambda j: (0, 0)),
            pl.BlockSpec((n, _BLK), lambda j: (0, j)),
        ],
        out_specs=pl.BlockSpec((1, 1), lambda j: (0, 0)),
        out_shape=jax.ShapeDtypeStruct((1, 1), jnp.float32),
        scratch_shapes=[pltpu.VMEM((n, 128), jnp.float32)],
    )(t2, x)
    return out[0, 0]


# speedup vs baseline: 1.0240x; 1.0240x over previous
import math

import jax
import jax.numpy as jnp
from jax.experimental import pallas as pl
from jax.experimental.pallas import tpu as pltpu

_SIZE = 32000
_PAD = 0
_SMOOTH = 0.1
_CONF = 1.0 - _SMOOTH
_FILL = _SMOOTH / (_SIZE - 2)
_C0 = _CONF * math.log(_CONF) + _SMOOTH * math.log(_FILL)
_K = _CONF / _FILL

_ROWS = 128


def _body(t_ref, x_ref, out_ref):
    j = pl.program_id(0)
    x = x_ref[...]
    r, size = x.shape
    t = t_ref[:, 0]

    p = jnp.zeros((r, 128), jnp.float32)
    for k in range(size // 128):
        xs = x[:, k * 128:(k + 1) * 128]
        cid = k * 128 + jax.lax.broadcasted_iota(jnp.int32, (r, 128), 1)
        z = jnp.where(cid == t[:, None], _K * xs, xs)
        if k == 0:
            z = jnp.where(cid == 0, 0.0, z)
        p = p + z

    ones = jnp.ones((128, 1), jnp.float32)
    rowz = jax.lax.dot(p, ones, preferred_element_type=jnp.float32)[:, 0]
    validf = (t != _PAD).astype(jnp.float32)
    partial = jnp.sum(validf) * _C0 - _FILL * jnp.sum(validf * rowz)

    @pl.when(j == 0)
    def _():
        out_ref[...] = partial.reshape(1, 1)

    @pl.when(j > 0)
    def _():
        out_ref[...] += partial.reshape(1, 1)


@jax.jit
def kernel(x, target):
    n, size = x.shape
    t2 = target.reshape(n, 1)
    grid = n // _ROWS
    out = pl.pallas_call(
        _body,
        grid=(grid,),
        in_specs=[
            pl.BlockSpec((_ROWS, 1), lambda j: (j, 0)),
            pl.BlockSpec((_ROWS, size), lambda j: (j, 0)),
        ],
        out_specs=pl.BlockSpec((1, 1), lambda j: (0, 0)),
        out_shape=jax.ShapeDtypeStruct((1, 1), jnp.float32),
    )(t2, x)
    return out[0, 0]
